# Initial kernel scaffold; baseline (speedup 1.0000x reference)
import jax
import jax.numpy as jnp
from jax.experimental import pallas as pl


def kernel(t, x, edge_index, W, b):
    raise NotImplementedError("write your pallas kernel here")


# speedup vs baseline: 17.7983x; 17.7983x over previous
import functools

import jax
import jax.numpy as jnp
from jax import lax
from jax.experimental import pallas as pl
from jax.experimental.pallas import tpu as pltpu
from jax.experimental.pallas import tpu_sc as plsc

N = 10000
NP = 10240
E = 320000
D = 128

NC = 2
NS = 16
NW = NC * NS
CH = 80
PER_W = E // NW
N_CH = PER_W // CH
RPT = NP // NS
CW = 16


def _mesh():
    return plsc.VectorSubcoreMesh(core_axis_name="c", subcore_axis_name="s")


@functools.partial(
    pl.kernel,
    out_type=jax.ShapeDtypeStruct((NC, NP), jnp.float32),
    mesh=_mesh(),
    scratch_types=[
        pltpu.VMEM((CH,), jnp.int32),
        pltpu.VMEM((CH,), jnp.float32),
        pltpu.VMEM_SHARED((NP,), jnp.float32),
    ],
)
def _sc_degree(dst_hbm, zeros_hbm, ones_hbm, cnt_hbm, idx_v, ones_v, cnt_sh):
    c = lax.axis_index("c")
    s = lax.axis_index("s")
    wid = s * NC + c
    pltpu.sync_copy(zeros_hbm.at[pl.ds(s * RPT, RPT)],
                    cnt_sh.at[pl.ds(s * RPT, RPT)])
    pltpu.sync_copy(ones_hbm, ones_v)
    plsc.subcore_barrier()
    base = wid * PER_W

    def body(i, carry):
        off = pl.multiple_of(base + i * CH, 8)
        pltpu.sync_copy(dst_hbm.at[pl.ds(off, CH)], idx_v)
        pltpu.sync_copy(ones_v, cnt_sh.at[idx_v], add=True)
        return carry

    lax.fori_loop(0, N_CH, body, 0)
    plsc.subcore_barrier()
    pltpu.sync_copy(cnt_sh.at[pl.ds(s * RPT, RPT)],
                    cnt_hbm.at[c, pl.ds(s * RPT, RPT)])


@functools.partial(
    pl.kernel,
    out_type=jax.ShapeDtypeStruct((NC, NP, D), jnp.float32),
    mesh=_mesh(),
    scratch_types=[
        pltpu.VMEM((CH,), jnp.int32),
        pltpu.VMEM((CH,), jnp.int32),
        pltpu.VMEM((CH, D), jnp.float32),
        pltpu.VMEM_SHARED((NP, D), jnp.float32),
        pltpu.SemaphoreType.DMA,
    ],
)
def _sc_edges(g_hbm, src_hbm, dst_hbm, zeros_hbm, p_hbm,
              src_v, dst_v, rows_v, acc_sh, sem):
    c = lax.axis_index("c")
    s = lax.axis_index("s")
    wid = s * NC + c
    pltpu.sync_copy(zeros_hbm.at[pl.ds(s * RPT, RPT)],
                    acc_sh.at[pl.ds(s * RPT, RPT)])
    plsc.subcore_barrier()
    base = wid * PER_W

    def body(i, carry):
        off = pl.multiple_of(base + i * CH, 8)
        pltpu.sync_copy(src_hbm.at[pl.ds(off, CH)], src_v)
        pltpu.sync_copy(dst_hbm.at[pl.ds(off, CH)], dst_v)
        pltpu.async_copy(g_hbm.at[src_v], rows_v, sem).wait()
        pltpu.sync_copy(rows_v, acc_sh.at[dst_v], add=True)
        return carry

    lax.fori_loop(0, N_CH, body, 0)
    plsc.subcore_barrier()
    pltpu.sync_copy(acc_sh.at[pl.ds(s * RPT, RPT)],
                    p_hbm.at[c, pl.ds(s * RPT, RPT)])


def _tc_prep_body(x_ref, w_ref, cnt_ref, g_ref, dinv_ref):
    h = jnp.dot(x_ref[...], w_ref[...], preferred_element_type=jnp.float32)
    deg = (cnt_ref[0, :] + cnt_ref[1, :] + 1.0)[:, None]
    dinv = lax.rsqrt(deg)
    dinv_ref[...] = dinv
    g_ref[...] = h * dinv


_tc_prep = pl.pallas_call(
    _tc_prep_body,
    out_shape=[
        jax.ShapeDtypeStruct((NP, D), jnp.float32),
        jax.ShapeDtypeStruct((NP, 1), jnp.float32),
    ],
)


def _tc_finish_body(p_ref, g_ref, dinv_ref, b_ref, o_ref):
    tot = p_ref[0] + p_ref[1] + g_ref[...]
    o_ref[...] = jnp.maximum(tot * dinv_ref[...] + b_ref[...], 0.0)


_tc_finish = pl.pallas_call(
    _tc_finish_body,
    out_shape=jax.ShapeDtypeStruct((NP, D), jnp.float32),
)


def kernel(t, x, edge_index, W, b):
    src = edge_index[0]
    dst = edge_index[1]
    xp = jnp.concatenate([x, jnp.zeros((NP - N, D), jnp.float32)], axis=0)
    zeros_cw = jnp.zeros((NP,), jnp.float32)
    ones_cw = jnp.ones((CH,), jnp.float32)
    zeros_nd = jnp.zeros((NP, D), jnp.float32)
    cnt = _sc_degree(dst, zeros_cw, ones_cw)
    g, dinv = _tc_prep(xp, W, cnt)
    p = _sc_edges(g, src, dst, zeros_nd)
    return _tc_finish(p, g, dinv, b.reshape(1, D))[:N]


# speedup vs baseline: 25.6060x; 1.4387x over previous
import functools

import jax
import jax.numpy as jnp
from jax import lax
from jax.experimental import pallas as pl
from jax.experimental.pallas import tpu as pltpu
from jax.experimental.pallas import tpu_sc as plsc

N = 10000
NP = 10240
E = 320000
D = 128

NC = 2
NS = 16
NW = NC * NS
PER_W = E // NW
CH = 128
NCHT = (PER_W + CH - 1) // CH
PAD = NCHT * CH - PER_W
RPT = NP // NS
NBUF = 4

DEG_GROUPS = ((0, 26), (26, 52), (52, NCHT))


def _mesh():
    return plsc.VectorSubcoreMesh(core_axis_name="c", subcore_axis_name="s")


@functools.partial(
    pl.kernel,
    out_type=jax.ShapeDtypeStruct((NC, NP), jnp.float32),
    mesh=_mesh(),
    scratch_types=[
        pltpu.VMEM((NCHT, CH), jnp.int32),
        pltpu.VMEM((CH,), jnp.float32),
        pltpu.VMEM_SHARED((NP,), jnp.float32),
        pltpu.SemaphoreType.DMA,
    ],
)
def _sc_degree(dstp_hbm, zeros_hbm, ones_hbm, cnt_hbm, dst_i, ones_v, cnt_sh,
               sem):
    c = lax.axis_index("c")
    s = lax.axis_index("s")
    wid = s * NC + c
    pltpu.sync_copy(zeros_hbm.at[pl.ds(s * RPT, RPT)],
                    cnt_sh.at[pl.ds(s * RPT, RPT)])
    pltpu.sync_copy(dstp_hbm.at[wid], dst_i)
    pltpu.sync_copy(ones_hbm, ones_v)
    plsc.subcore_barrier()

    for lo, hi in DEG_GROUPS:
        def fire(i, carry):
            pltpu.async_copy(ones_v, cnt_sh.at[dst_i.at[i]], sem, add=True)
            return carry

        lax.fori_loop(lo, hi, fire, 0)

        def drain(i, carry):
            pltpu.make_async_copy(ones_v, cnt_sh.at[dst_i.at[i]], sem).wait()
            return carry

        lax.fori_loop(lo, hi, drain, 0)

    plsc.subcore_barrier()
    pltpu.sync_copy(cnt_sh.at[pl.ds(s * RPT, RPT)],
                    cnt_hbm.at[c, pl.ds(s * RPT, RPT)])


@functools.partial(
    pl.kernel,
    out_type=jax.ShapeDtypeStruct((NC, NP, D), jnp.float32),
    mesh=_mesh(),
    scratch_types=[
        pltpu.VMEM((NCHT, CH), jnp.int32),
        pltpu.VMEM((1, CH), jnp.int32),
        pltpu.VMEM((1, CH), jnp.int32),
        pltpu.VMEM((1, CH), jnp.int32),
        pltpu.VMEM((1, CH), jnp.int32),
        pltpu.VMEM((CH, D), jnp.float32),
        pltpu.VMEM((CH, D), jnp.float32),
        pltpu.VMEM_SHARED((NP, D), jnp.float32),
        pltpu.SemaphoreType.DMA,
    ],
)
def _sc_edges(g_hbm, sd_hbm, zeros_hbm, p_hbm,
              sd_v, src_a, src_b, dst_a, dst_b, rows_a, rows_b, acc_sh, sem_g):
    c = lax.axis_index("c")
    s = lax.axis_index("s")
    wid = s * NC + c
    pltpu.sync_copy(zeros_hbm.at[pl.ds(s * RPT, RPT)],
                    acc_sh.at[pl.ds(s * RPT, RPT)])
    pltpu.sync_copy(sd_hbm.at[wid], sd_v)
    plsc.subcore_barrier()

    def widen(j, src_v, dst_v):
        for k in range(CH // 16):
            v = sd_v[j, pl.ds(k * 16, 16)]
            src_v[0, pl.ds(k * 16, 16)] = v & 0xFFFF
            dst_v[0, pl.ds(k * 16, 16)] = v >> 16

    def fire(src_v, rows_v):
        pltpu.async_copy(g_hbm.at[src_v.at[0]], rows_v, sem_g)

    def wait(src_v, rows_v):
        pltpu.make_async_copy(g_hbm.at[src_v.at[0]], rows_v, sem_g).wait()

    def scat(dst_v, rows_v):
        pltpu.sync_copy(rows_v, acc_sh.at[dst_v.at[0]], add=True)

    widen(0, src_a, dst_a)
    fire(src_a, rows_a)
    widen(1, src_b, dst_b)
    fire(src_b, rows_b)

    def body(jj, carry):
        i = 2 * jj
        wait(src_a, rows_a)
        scat(dst_a, rows_a)
        widen(i + 2, src_a, dst_a)
        fire(src_a, rows_a)
        wait(src_b, rows_b)
        scat(dst_b, rows_b)
        widen(i + 3, src_b, dst_b)
        fire(src_b, rows_b)
        return carry

    lax.fori_loop(0, (NCHT - 3) // 2, body, 0)

    wait(src_a, rows_a)
    scat(dst_a, rows_a)
    widen(NCHT - 1, src_a, dst_a)
    fire(src_a, rows_a)
    wait(src_b, rows_b)
    scat(dst_b, rows_b)
    wait(src_a, rows_a)
    scat(dst_a, rows_a)

    plsc.subcore_barrier()
    pltpu.sync_copy(acc_sh.at[pl.ds(s * RPT, RPT)],
                    p_hbm.at[c, pl.ds(s * RPT, RPT)])


def _tc_prep_body(x_ref, w_ref, cnt_ref, g_ref, dinv_ref):
    h = jnp.dot(x_ref[...], w_ref[...], preferred_element_type=jnp.float32)
    deg = (cnt_ref[0] + cnt_ref[1] + 1.0)[:, None]
    dinv = lax.rsqrt(deg)
    dinv_ref[...] = dinv
    g_ref[pl.ds(0, N)] = h * dinv[:N]


_tc_prep = pl.pallas_call(
    _tc_prep_body,
    out_shape=[
        jax.ShapeDtypeStruct((NP, D), jnp.float32),
        jax.ShapeDtypeStruct((NP, 1), jnp.float32),
    ],
)


def _tc_finish_body(p_ref, g_ref, dinv_ref, b_ref, o_ref):
    tot = p_ref[0, :N] + p_ref[1, :N] + g_ref[:N]
    o_ref[...] = jnp.maximum(tot * dinv_ref[:N] + b_ref[...], 0.0)


_tc_finish = pl.pallas_call(
    _tc_finish_body,
    out_shape=jax.ShapeDtypeStruct((N, D), jnp.float32),
)


def kernel(t, x, edge_index, W, b):
    src2 = edge_index[0].reshape(NW, PER_W)
    dst2 = edge_index[1].reshape(NW, PER_W)
    dst_p = jnp.concatenate(
        [dst2, jnp.full((NW, PAD), N + 16, jnp.int32)],
        axis=1).reshape(NW, NCHT, CH)
    sd2 = src2 | (dst2 << 16)
    sd_p = jnp.concatenate(
        [sd2, jnp.full((NW, PAD), (N + 16) << 16, jnp.int32)],
        axis=1).reshape(NW, NCHT, CH)
    zeros_np = jnp.zeros((NP,), jnp.float32)
    ones_ch = jnp.ones((CH,), jnp.float32)
    zeros_nd = jnp.zeros((NP, D), jnp.float32)
    cnt = _sc_degree(dst_p, zeros_np, ones_ch)
    g, dinv = _tc_prep(x, W, cnt)
    p = _sc_edges(g, sd_p, zeros_nd)
    return _tc_finish(p, g, dinv, b.reshape(1, D))


# speedup vs baseline: 46.8556x; 1.8299x over previous
import functools

import jax
import jax.numpy as jnp
from jax import lax
from jax.experimental import pallas as pl
from jax.experimental.pallas import tpu as pltpu
from jax.experimental.pallas import tpu_sc as plsc

N = 10000
NP = 10240
E = 320000
D = 128

NC = 2
NS = 16
NW = NC * NS
PER_W = E // NW
CH = 128
NCHT = (PER_W + CH - 1) // CH
PAD = NCHT * CH - PER_W
CHE = 80
NCHE = PER_W // CHE
RPT = NP // NS

DEG_GROUPS = ((0, 26), (26, 52), (52, NCHT))


def _mesh():
    return plsc.VectorSubcoreMesh(core_axis_name="c", subcore_axis_name="s")


@functools.partial(
    pl.kernel,
    out_type=jax.ShapeDtypeStruct((NC, NP), jnp.float32),
    mesh=_mesh(),
    scratch_types=[
        pltpu.VMEM((NCHT, CH), jnp.int32),
        pltpu.VMEM((CH,), jnp.float32),
        pltpu.VMEM_SHARED((NP,), jnp.float32),
        pltpu.SemaphoreType.DMA,
    ],
)
def _sc_degree(dstp_hbm, zeros_hbm, ones_hbm, cnt_hbm, dst_i, ones_v, cnt_sh,
               sem):
    c = lax.axis_index("c")
    s = lax.axis_index("s")
    wid = s * NC + c
    pltpu.sync_copy(zeros_hbm.at[pl.ds(s * RPT, RPT)],
                    cnt_sh.at[pl.ds(s * RPT, RPT)])
    pltpu.sync_copy(dstp_hbm.at[wid], dst_i)
    pltpu.sync_copy(ones_hbm, ones_v)
    plsc.subcore_barrier()

    for lo, hi in DEG_GROUPS:
        def fire(i, carry):
            pltpu.async_copy(ones_v, cnt_sh.at[dst_i.at[i]], sem, add=True)
            return carry

        lax.fori_loop(lo, hi, fire, 0)

        def drain(i, carry):
            pltpu.make_async_copy(ones_v, cnt_sh.at[dst_i.at[i]], sem).wait()
            return carry

        lax.fori_loop(lo, hi, drain, 0)

    plsc.subcore_barrier()
    pltpu.sync_copy(cnt_sh.at[pl.ds(s * RPT, RPT)],
                    cnt_hbm.at[c, pl.ds(s * RPT, RPT)])


@functools.partial(
    pl.kernel,
    out_type=jax.ShapeDtypeStruct((NC, NP, D), jnp.float32),
    mesh=_mesh(),
    scratch_types=[
        pltpu.VMEM((PER_W,), jnp.int32),
        pltpu.VMEM((8, CHE), jnp.int32),
        pltpu.VMEM((CHE, D), jnp.float32),
        pltpu.VMEM((CHE, D), jnp.float32),
        pltpu.VMEM((CHE, D), jnp.float32),
        pltpu.VMEM_SHARED((NP, D), jnp.float32),
        pltpu.SemaphoreType.DMA,
        pltpu.SemaphoreType.DMA,
    ],
)
def _sc_edges(g_hbm, sd_hbm, zeros_hbm, p_hbm,
              sd_v, sl, rows_a, rows_b, rows_c, acc_sh, sem_g, sem_s):
    c = lax.axis_index("c")
    s = lax.axis_index("s")
    wid = s * NC + c
    pltpu.sync_copy(zeros_hbm.at[pl.ds(s * RPT, RPT)],
                    acc_sh.at[pl.ds(s * RPT, RPT)])
    pltpu.sync_copy(sd_hbm.at[pl.ds(wid * PER_W, PER_W)], sd_v)
    plsc.subcore_barrier()

    rows_ = (rows_a, rows_b, rows_c)

    def widen(j, k):
        for m in range(CHE // 16):
            v = sd_v[pl.ds(j * CHE + m * 16, 16)]
            sl[k, pl.ds(m * 16, 16)] = v & 0xFFFF
            sl[k + 4, pl.ds(m * 16, 16)] = v >> 16

    def fire_g(k):
        pltpu.async_copy(g_hbm.at[sl.at[k]], rows_[k], sem_g)

    def wait_g(k):
        pltpu.make_async_copy(g_hbm.at[sl.at[k]], rows_[k], sem_g).wait()

    def fire_s(k):
        pltpu.async_copy(rows_[k], acc_sh.at[sl.at[k + 4]], sem_s, add=True)

    def wait_s(k):
        pltpu.make_async_copy(rows_[k], acc_sh.at[sl.at[k + 4]], sem_s).wait()

    widen(0, 0)
    fire_g(0)
    widen(1, 1)
    fire_g(1)
    wait_g(0)
    fire_s(0)
    widen(2, 2)
    fire_g(2)
    wait_g(1)
    fire_s(1)

    def body(j, carry):
        for o in range(3):
            i = 3 * j + 2 + o
            kf = (2 + o) % 3
            kw = o % 3
            wait_s(kw)
            widen(i + 1, kw)
            fire_g(kw)
            wait_g(kf)
            fire_s(kf)
        return carry

    lax.fori_loop(0, (NCHE - 5) // 3, body, 0)

    wait_s(0)
    widen(NCHE - 2, 0)
    fire_g(0)
    wait_g(2)
    fire_s(2)
    wait_s(1)
    widen(NCHE - 1, 1)
    fire_g(1)
    wait_g(0)
    fire_s(0)
    wait_s(2)
    wait_g(1)
    fire_s(1)
    wait_s(0)
    wait_s(1)

    plsc.subcore_barrier()
    pltpu.sync_copy(acc_sh.at[pl.ds(s * RPT, RPT)],
                    p_hbm.at[c, pl.ds(s * RPT, RPT)])


def _tc_prep_body(x_ref, w_ref, cnt_ref, g_ref, dinv_ref):
    h = jnp.dot(x_ref[...], w_ref[...], preferred_element_type=jnp.float32)
    deg = (cnt_ref[0] + cnt_ref[1] + 1.0)[:, None]
    dinv = lax.rsqrt(deg)
    dinv_ref[...] = dinv
    g_ref[pl.ds(0, N)] = h * dinv[:N]


_tc_prep = pl.pallas_call(
    _tc_prep_body,
    out_shape=[
        jax.ShapeDtypeStruct((NP, D), jnp.float32),
        jax.ShapeDtypeStruct((NP, 1), jnp.float32),
    ],
)


def _tc_finish_body(p_ref, g_ref, dinv_ref, b_ref, o_ref):
    tot = p_ref[0, :N] + p_ref[1, :N] + g_ref[:N]
    o_ref[...] = jnp.maximum(tot * dinv_ref[:N] + b_ref[...], 0.0)


_tc_finish = pl.pallas_call(
    _tc_finish_body,
    out_shape=jax.ShapeDtypeStruct((N, D), jnp.float32),
)


def kernel(t, x, edge_index, W, b):
    src2 = edge_index[0].reshape(NW, PER_W)
    dst2 = edge_index[1].reshape(NW, PER_W)
    dst_p = jnp.concatenate(
        [dst2, jnp.full((NW, PAD), N + 16, jnp.int32)],
        axis=1).reshape(NW, NCHT, CH)
    sd_p = (src2 | (dst2 << 16)).reshape(E)
    zeros_np = jnp.zeros((NP,), jnp.float32)
    ones_ch = jnp.ones((CH,), jnp.float32)
    zeros_nd = jnp.zeros((NP, D), jnp.float32)
    cnt = _sc_degree(dst_p, zeros_np, ones_ch)
    g, dinv = _tc_prep(x, W, cnt)
    p = _sc_edges(g, sd_p, zeros_nd)
    return _tc_finish(p, g, dinv, b.reshape(1, D))
